# baseline (speedup 1.0000x reference)
import functools

import jax
import jax.numpy as jnp
from jax import lax
from jax.experimental import pallas as pl
from jax.experimental.pallas import tpu as pltpu
from jax.experimental.pallas import tpu_sc as plsc

_N = 10000
_D = 128
_E = 320000

_NC = 2
_NS = 16
_NW = _NC * _NS
_C = 80
_NCH = 128
_EPW = _NCH * _C
_EP = _NW * _EPW
_NA = 10240
_RPS = _NA // _NS
_LAST = _N - 15 * _RPS
_PAD_DST = _NA - 8

_TCB = 2000
_TCG = _N // _TCB


def _make_sc_degree():
  mesh = plsc.VectorSubcoreMesh(core_axis_name="c", subcore_axis_name="s")

  @functools.partial(
      pl.kernel,
      out_type=jax.ShapeDtypeStruct((_NC * _NA,), jnp.float32),
      mesh=mesh,
      scratch_types=[
          pltpu.VMEM((_NCH, _C), jnp.int32),
          pltpu.VMEM((_C,), jnp.float32),
          pltpu.VMEM((_RPS,), jnp.float32),
          pltpu.VMEM_SHARED((_NA,), jnp.float32),
      ],
  )
  def deg_kernel(dst_hbm, out_hbm, didx_v, ones_v, zb_v, acc_sh):
    cid = lax.axis_index("c")
    sid = lax.axis_index("s")
    wid = cid * _NS + sid

    def fill(r, carry):
      ones_v[pl.ds(r * 16, 16)] = jnp.ones((16,), jnp.float32)
      return carry

    lax.fori_loop(0, _C // 16, fill, 0)

    def zfill(r, carry):
      zb_v[pl.ds(r * 16, 16)] = jnp.zeros((16,), jnp.float32)
      return carry

    lax.fori_loop(0, _RPS // 16, zfill, 0)
    pltpu.sync_copy(zb_v, acc_sh.at[pl.ds(sid * _RPS, _RPS)])
    pltpu.sync_copy(dst_hbm.at[wid], didx_v)
    plsc.subcore_barrier()

    def chunk(i, carry):
      pltpu.sync_copy(ones_v, acc_sh.at[didx_v.at[i]], add=True)
      return carry

    lax.fori_loop(0, _NCH, chunk, 0)
    plsc.subcore_barrier()
    pltpu.sync_copy(acc_sh.at[pl.ds(sid * _RPS, _RPS)],
                    out_hbm.at[pl.ds(cid * _NA + sid * _RPS, _RPS)])

  return deg_kernel


def _make_sc_scatter():
  mesh = plsc.VectorSubcoreMesh(core_axis_name="c", subcore_axis_name="s")

  @functools.partial(
      pl.kernel,
      out_type=jax.ShapeDtypeStruct((_NC, _N, _D), jnp.float32),
      mesh=mesh,
      scratch_types=[
          pltpu.VMEM((_C,), jnp.int32),
          pltpu.VMEM((_C,), jnp.int32),
          pltpu.VMEM((_C,), jnp.int32),
          pltpu.VMEM((_C,), jnp.int32),
          pltpu.VMEM((_C, _D), jnp.float32),
          pltpu.VMEM((_C, _D), jnp.float32),
          pltpu.VMEM_SHARED((_NA, _D), jnp.float32),
          pltpu.SemaphoreType.DMA,
          pltpu.SemaphoreType.DMA,
          pltpu.SemaphoreType.DMA,
          pltpu.SemaphoreType.DMA,
      ],
  )
  def scat_kernel(p_hbm, src_hbm, dst_hbm, out_hbm, sidx0, didx0, sidx1,
                  didx1, rows0, rows1, acc_sh, g0, g1, i0, i1):
    cid = lax.axis_index("c")
    sid = lax.axis_index("s")
    wid = cid * _NS + sid
    base = wid * _EPW

    def zfill(r, carry):
      for cc in range(_D // 16):
        rows0[r, pl.ds(cc * 16, 16)] = jnp.zeros((16,), jnp.float32)
      return carry

    lax.fori_loop(0, _C, zfill, 0)

    def zcopy(j, carry):
      pltpu.sync_copy(rows0, acc_sh.at[pl.ds(sid * _RPS + j * _C, _C)])
      return carry

    lax.fori_loop(0, _RPS // _C, zcopy, 0)
    plsc.subcore_barrier()

    def chunk(i, carry):
      pltpu.sync_copy(src_hbm.at[pl.ds(base + i * _C, _C)], sidx0)
      pltpu.sync_copy(dst_hbm.at[pl.ds(base + i * _C, _C)], didx0)
      pltpu.async_copy(p_hbm.at[sidx0], rows0, g0).wait()
      pltpu.sync_copy(rows0, acc_sh.at[didx0], add=True)
      return carry

    lax.fori_loop(0, _NCH, chunk, 0)
    plsc.subcore_barrier()

    @pl.when(sid < _NS - 1)
    def _full():
      pltpu.sync_copy(
          acc_sh.at[pl.ds(sid * _RPS, _RPS)],
          out_hbm.at[cid, pl.ds(sid * _RPS, _RPS)],
      )

    @pl.when(sid == _NS - 1)
    def _tail():
      pltpu.sync_copy(
          acc_sh.at[pl.ds(sid * _RPS, _LAST)],
          out_hbm.at[cid, pl.ds(sid * _RPS, _LAST)],
      )

  return scat_kernel


def _dinv_block(dpt_ref):
  deg = dpt_ref[:, 0:1] + dpt_ref[:, 1:2] + 1.0
  return lax.rsqrt(deg)


def _tc_first(x, w, dpt):
  def body(x_ref, w_ref, dpt_ref, p_ref):
    dinv = _dinv_block(dpt_ref)
    h = jnp.dot(x_ref[...], w_ref[...], preferred_element_type=jnp.float32)
    p_ref[...] = h * dinv

  return pl.pallas_call(
      body,
      grid=(_TCG,),
      in_specs=[
          pl.BlockSpec((_TCB, _D), lambda i: (i, 0)),
          pl.BlockSpec((_D, _D), lambda i: (0, 0)),
          pl.BlockSpec((_TCB, _NC), lambda i: (i, 0)),
      ],
      out_specs=pl.BlockSpec((_TCB, _D), lambda i: (i, 0)),
      out_shape=jax.ShapeDtypeStruct((_N, _D), jnp.float32),
  )(x, w, dpt)


def _tc_mid(p, dpt, sp, w, b):
  def body(p_ref, dpt_ref, sp_ref, w_ref, b_ref, o_ref):
    dinv = _dinv_block(dpt_ref)
    s = sp_ref[0] + sp_ref[1] + p_ref[...]
    a = jnp.maximum(s * dinv + b_ref[...], 0.0)
    h = jnp.dot(a, w_ref[...], preferred_element_type=jnp.float32)
    o_ref[...] = h * dinv

  return pl.pallas_call(
      body,
      grid=(_TCG,),
      in_specs=[
          pl.BlockSpec((_TCB, _D), lambda i: (i, 0)),
          pl.BlockSpec((_TCB, _NC), lambda i: (i, 0)),
          pl.BlockSpec((_NC, _TCB, _D), lambda i: (0, i, 0)),
          pl.BlockSpec((_D, _D), lambda i: (0, 0)),
          pl.BlockSpec((1, _D), lambda i: (0, 0)),
      ],
      out_specs=pl.BlockSpec((_TCB, _D), lambda i: (i, 0)),
      out_shape=jax.ShapeDtypeStruct((_N, _D), jnp.float32),
  )(p, dpt, sp, w, b)


def _tc_final(p, dpt, sp, b):
  def body(p_ref, dpt_ref, sp_ref, b_ref, o_ref):
    dinv = _dinv_block(dpt_ref)
    s = sp_ref[0] + sp_ref[1] + p_ref[...]
    o_ref[...] = s * dinv + b_ref[...]

  return pl.pallas_call(
      body,
      grid=(_TCG,),
      in_specs=[
          pl.BlockSpec((_TCB, _D), lambda i: (i, 0)),
          pl.BlockSpec((_TCB, _NC), lambda i: (i, 0)),
          pl.BlockSpec((_NC, _TCB, _D), lambda i: (0, i, 0)),
          pl.BlockSpec((1, _D), lambda i: (0, 0)),
      ],
      out_specs=pl.BlockSpec((_TCB, _D), lambda i: (i, 0)),
      out_shape=jax.ShapeDtypeStruct((_N, _D), jnp.float32),
  )(p, dpt, sp, b)


_sc_degree = _make_sc_degree()
_sc_scatter = _make_sc_scatter()


def kernel(x, edge_index, W1, b1, W2, b2, W3, b3):
  npad = _EP - _E
  pad_dst = _N + jax.lax.rem(jnp.arange(npad, dtype=edge_index.dtype),
                             jnp.asarray(_NA - _N, edge_index.dtype))
  pad = jnp.stack([jnp.zeros((npad,), edge_index.dtype), pad_dst])
  ep = jnp.concatenate([edge_index, pad], axis=1)
  src1 = ep[0]
  dst1 = ep[1]
  dst3 = dst1.reshape(_NW, _NCH, _C)
  b1r = b1.reshape(1, _D)
  b2r = b2.reshape(1, _D)
  b3r = b3.reshape(1, _D)

  dflat = _sc_degree(dst3).reshape(_NC, _NA)
  dpt = jnp.stack([dflat[0, :_N], dflat[1, :_N]], axis=1)
  p1 = _tc_first(x, W1, dpt)
  sp = _sc_scatter(p1, src1, dst1)
  p2 = _tc_mid(p1, dpt, sp, W2, b1r)
  sp = _sc_scatter(p2, src1, dst1)
  p3 = _tc_mid(p2, dpt, sp, W3, b2r)
  sp = _sc_scatter(p3, src1, dst1)
  return _tc_final(p3, dpt, sp, b3r)


# speedup vs baseline: 2.1372x; 2.1372x over previous
import functools

import jax
import jax.numpy as jnp
from jax import lax
from jax.experimental import pallas as pl
from jax.experimental.pallas import tpu as pltpu
from jax.experimental.pallas import tpu_sc as plsc

_N = 10000
_D = 128
_E = 320000

_NC = 2
_NS = 16
_NW = _NC * _NS
_C = 80
_NCH = 125
_EPW = _NCH * _C
_EP = _NW * _EPW
_NA = 10240
_RPS = _NA // _NS
_LAST = _N - 15 * _RPS
_PAD_DST = _NA - 8

_TCB = 2000
_TCG = _N // _TCB


def _make_sc_degree():
  mesh = plsc.VectorSubcoreMesh(core_axis_name="c", subcore_axis_name="s")

  @functools.partial(
      pl.kernel,
      out_type=jax.ShapeDtypeStruct((_NC * _NA,), jnp.float32),
      mesh=mesh,
      scratch_types=[
          pltpu.VMEM((_NCH, _C), jnp.int32),
          pltpu.VMEM((_C,), jnp.float32),
          pltpu.VMEM((_RPS,), jnp.float32),
          pltpu.VMEM_SHARED((_NA,), jnp.float32),
      ],
  )
  def deg_kernel(dst_hbm, out_hbm, didx_v, ones_v, zb_v, acc_sh):
    cid = lax.axis_index("c")
    sid = lax.axis_index("s")
    wid = cid * _NS + sid

    def fill(r, carry):
      ones_v[pl.ds(r * 16, 16)] = jnp.ones((16,), jnp.float32)
      return carry

    lax.fori_loop(0, _C // 16, fill, 0)

    def zfill(r, carry):
      zb_v[pl.ds(r * 16, 16)] = jnp.zeros((16,), jnp.float32)
      return carry

    lax.fori_loop(0, _RPS // 16, zfill, 0)
    pltpu.sync_copy(zb_v, acc_sh.at[pl.ds(sid * _RPS, _RPS)])
    pltpu.sync_copy(dst_hbm.at[wid], didx_v)
    plsc.subcore_barrier()

    def chunk(i, carry):
      pltpu.sync_copy(ones_v, acc_sh.at[didx_v.at[i]], add=True)
      return carry

    lax.fori_loop(0, _NCH, chunk, 0)
    plsc.subcore_barrier()
    pltpu.sync_copy(acc_sh.at[pl.ds(sid * _RPS, _RPS)],
                    out_hbm.at[pl.ds(cid * _NA + sid * _RPS, _RPS)])

  return deg_kernel


def _make_sc_scatter():
  mesh = plsc.VectorSubcoreMesh(core_axis_name="c", subcore_axis_name="s")

  @functools.partial(
      pl.kernel,
      out_type=jax.ShapeDtypeStruct((_NC, _N, _D), jnp.float32),
      mesh=mesh,
      scratch_types=[
          pltpu.VMEM((_C,), jnp.int32),
          pltpu.VMEM((_C,), jnp.int32),
          pltpu.VMEM((_C,), jnp.int32),
          pltpu.VMEM((_C,), jnp.int32),
          pltpu.VMEM((_C, _D), jnp.float32),
          pltpu.VMEM((_C, _D), jnp.float32),
          pltpu.VMEM_SHARED((_NA, _D), jnp.float32),
          pltpu.SemaphoreType.DMA,
          pltpu.SemaphoreType.DMA,
          pltpu.SemaphoreType.DMA,
          pltpu.SemaphoreType.DMA,
      ],
  )
  def scat_kernel(p_hbm, src_hbm, dst_hbm, out_hbm, sidx0, didx0, sidx1,
                  didx1, rows0, rows1, acc_sh, g0, g1, i0, i1):
    cid = lax.axis_index("c")
    sid = lax.axis_index("s")
    wid = cid * _NS + sid
    base = wid * _EPW

    def zfill(r, carry):
      for cc in range(_D // 16):
        rows0[r, pl.ds(cc * 16, 16)] = jnp.zeros((16,), jnp.float32)
      return carry

    lax.fori_loop(0, _C, zfill, 0)

    def zcopy(j, carry):
      pltpu.sync_copy(rows0, acc_sh.at[pl.ds(sid * _RPS + j * _C, _C)])
      return carry

    lax.fori_loop(0, _RPS // _C, zcopy, 0)
    plsc.subcore_barrier()

    def chunk(i, carry):
      pltpu.sync_copy(src_hbm.at[pl.ds(base + i * _C, _C)], sidx0)
      pltpu.sync_copy(dst_hbm.at[pl.ds(base + i * _C, _C)], didx0)
      pltpu.async_copy(p_hbm.at[sidx0], rows0, g0).wait()
      pltpu.sync_copy(rows0, acc_sh.at[didx0], add=True)
      return carry

    lax.fori_loop(0, _NCH, chunk, 0)
    plsc.subcore_barrier()

    @pl.when(sid < _NS - 1)
    def _full():
      pltpu.sync_copy(
          acc_sh.at[pl.ds(sid * _RPS, _RPS)],
          out_hbm.at[cid, pl.ds(sid * _RPS, _RPS)],
      )

    @pl.when(sid == _NS - 1)
    def _tail():
      pltpu.sync_copy(
          acc_sh.at[pl.ds(sid * _RPS, _LAST)],
          out_hbm.at[cid, pl.ds(sid * _RPS, _LAST)],
      )

  return scat_kernel


def _dinv_block(dpt_ref):
  deg = dpt_ref[:, 0:1] + dpt_ref[:, 1:2] + 1.0
  return lax.rsqrt(deg)


def _tc_first(x, w, dpt):
  def body(x_ref, w_ref, dpt_ref, p_ref):
    dinv = _dinv_block(dpt_ref)
    h = jnp.dot(x_ref[...], w_ref[...], preferred_element_type=jnp.float32)
    p_ref[...] = h * dinv

  return pl.pallas_call(
      body,
      grid=(_TCG,),
      in_specs=[
          pl.BlockSpec((_TCB, _D), lambda i: (i, 0)),
          pl.BlockSpec((_D, _D), lambda i: (0, 0)),
          pl.BlockSpec((_TCB, _NC), lambda i: (i, 0)),
      ],
      out_specs=pl.BlockSpec((_TCB, _D), lambda i: (i, 0)),
      out_shape=jax.ShapeDtypeStruct((_N, _D), jnp.float32),
  )(x, w, dpt)


def _tc_mid(p, dpt, sp, w, b):
  def body(p_ref, dpt_ref, sp_ref, w_ref, b_ref, o_ref):
    dinv = _dinv_block(dpt_ref)
    s = sp_ref[0] + sp_ref[1] + p_ref[...]
    a = jnp.maximum(s * dinv + b_ref[...], 0.0)
    h = jnp.dot(a, w_ref[...], preferred_element_type=jnp.float32)
    o_ref[...] = h * dinv

  return pl.pallas_call(
      body,
      grid=(_TCG,),
      in_specs=[
          pl.BlockSpec((_TCB, _D), lambda i: (i, 0)),
          pl.BlockSpec((_TCB, _NC), lambda i: (i, 0)),
          pl.BlockSpec((_NC, _TCB, _D), lambda i: (0, i, 0)),
          pl.BlockSpec((_D, _D), lambda i: (0, 0)),
          pl.BlockSpec((1, _D), lambda i: (0, 0)),
      ],
      out_specs=pl.BlockSpec((_TCB, _D), lambda i: (i, 0)),
      out_shape=jax.ShapeDtypeStruct((_N, _D), jnp.float32),
  )(p, dpt, sp, w, b)


def _tc_final(p, dpt, sp, b):
  def body(p_ref, dpt_ref, sp_ref, b_ref, o_ref):
    dinv = _dinv_block(dpt_ref)
    s = sp_ref[0] + sp_ref[1] + p_ref[...]
    o_ref[...] = s * dinv + b_ref[...]

  return pl.pallas_call(
      body,
      grid=(_TCG,),
      in_specs=[
          pl.BlockSpec((_TCB, _D), lambda i: (i, 0)),
          pl.BlockSpec((_TCB, _NC), lambda i: (i, 0)),
          pl.BlockSpec((_NC, _TCB, _D), lambda i: (0, i, 0)),
          pl.BlockSpec((1, _D), lambda i: (0, 0)),
      ],
      out_specs=pl.BlockSpec((_TCB, _D), lambda i: (i, 0)),
      out_shape=jax.ShapeDtypeStruct((_N, _D), jnp.float32),
  )(p, dpt, sp, b)


_sc_degree = _make_sc_degree()
_sc_scatter = _make_sc_scatter()


def kernel(x, edge_index, W1, b1, W2, b2, W3, b3):
  src1 = edge_index[0]
  dst1 = edge_index[1]
  dst3 = dst1.reshape(_NW, _NCH, _C)
  b1r = b1.reshape(1, _D)
  b2r = b2.reshape(1, _D)
  b3r = b3.reshape(1, _D)

  dflat = _sc_degree(dst3).reshape(_NC, _NA)
  dpt = jnp.stack([dflat[0, :_N], dflat[1, :_N]], axis=1)
  p1 = _tc_first(x, W1, dpt)
  sp = _sc_scatter(p1, src1, dst1)
  p2 = _tc_mid(p1, dpt, sp, W2, b1r)
  sp = _sc_scatter(p2, src1, dst1)
  p3 = _tc_mid(p2, dpt, sp, W3, b2r)
  sp = _sc_scatter(p3, src1, dst1)
  return _tc_final(p3, dpt, sp, b3r)


# speedup vs baseline: 4.1314x; 1.9331x over previous
import functools

import jax
import jax.numpy as jnp
from jax import lax
from jax.experimental import pallas as pl
from jax.experimental.pallas import tpu as pltpu
from jax.experimental.pallas import tpu_sc as plsc

_N = 10000
_D = 128
_E = 320000

_NC = 2
_NS = 16
_NW = _NC * _NS
_C = 80
_NCH = 125
_EPW = _NCH * _C
_EP = _NW * _EPW
_NA = 10240
_RPS = _NA // _NS
_LAST = _N - 15 * _RPS
_PAD_DST = _NA - 8

_TCB = 2000
_TCG = _N // _TCB


def _make_sc_degree():
  mesh = plsc.VectorSubcoreMesh(core_axis_name="c", subcore_axis_name="s")

  @functools.partial(
      pl.kernel,
      out_type=jax.ShapeDtypeStruct((_NC * _NA,), jnp.float32),
      mesh=mesh,
      scratch_types=[
          pltpu.VMEM((_NCH, _C), jnp.int32),
          pltpu.VMEM((_C,), jnp.float32),
          pltpu.VMEM((_RPS,), jnp.float32),
          pltpu.VMEM_SHARED((_NA,), jnp.float32),
      ],
  )
  def deg_kernel(dst_hbm, out_hbm, didx_v, ones_v, zb_v, acc_sh):
    cid = lax.axis_index("c")
    sid = lax.axis_index("s")
    wid = cid * _NS + sid

    def fill(r, carry):
      ones_v[pl.ds(r * 16, 16)] = jnp.ones((16,), jnp.float32)
      return carry

    lax.fori_loop(0, _C // 16, fill, 0)

    def zfill(r, carry):
      zb_v[pl.ds(r * 16, 16)] = jnp.zeros((16,), jnp.float32)
      return carry

    lax.fori_loop(0, _RPS // 16, zfill, 0)
    pltpu.sync_copy(zb_v, acc_sh.at[pl.ds(sid * _RPS, _RPS)])
    pltpu.sync_copy(dst_hbm.at[wid], didx_v)
    plsc.subcore_barrier()

    def chunk(i, carry):
      pltpu.sync_copy(ones_v, acc_sh.at[didx_v.at[i]], add=True)
      return carry

    lax.fori_loop(0, _NCH, chunk, 0)
    plsc.subcore_barrier()
    pltpu.sync_copy(acc_sh.at[pl.ds(sid * _RPS, _RPS)],
                    out_hbm.at[pl.ds(cid * _NA + sid * _RPS, _RPS)])

  return deg_kernel


def _make_sc_scatter():
  mesh = plsc.VectorSubcoreMesh(core_axis_name="c", subcore_axis_name="s")

  @functools.partial(
      pl.kernel,
      out_type=jax.ShapeDtypeStruct((_NC, _N, _D), jnp.float32),
      mesh=mesh,
      scratch_types=[
          pltpu.VMEM((_C,), jnp.int32),
          pltpu.VMEM((_C,), jnp.int32),
          pltpu.VMEM((_C,), jnp.int32),
          pltpu.VMEM((_C,), jnp.int32),
          pltpu.VMEM((_C, _D), jnp.float32),
          pltpu.VMEM((_C, _D), jnp.float32),
          pltpu.VMEM_SHARED((_NA, _D), jnp.float32),
          pltpu.SemaphoreType.DMA,
          pltpu.SemaphoreType.DMA,
          pltpu.SemaphoreType.DMA,
          pltpu.SemaphoreType.DMA,
      ],
  )
  def scat_kernel(p_hbm, src_hbm, dst_hbm, out_hbm, sidx0, didx0, sidx1,
                  didx1, rows0, rows1, acc_sh, g0, g1, i0, i1):
    cid = lax.axis_index("c")
    sid = lax.axis_index("s")
    wid = cid * _NS + sid
    base = wid * _EPW

    def zfill(r, carry):
      for cc in range(_D // 16):
        rows0[r, pl.ds(cc * 16, 16)] = jnp.zeros((16,), jnp.float32)
      return carry

    lax.fori_loop(0, _C, zfill, 0)

    def zcopy(j, carry):
      pltpu.sync_copy(rows0, acc_sh.at[pl.ds(sid * _RPS + j * _C, _C)])
      return carry

    lax.fori_loop(0, _RPS // _C, zcopy, 0)
    plsc.subcore_barrier()

    pltpu.sync_copy(src_hbm.at[pl.ds(base, _C)], sidx0)
    pltpu.sync_copy(dst_hbm.at[pl.ds(base, _C)], didx0)
    pltpu.sync_copy(src_hbm.at[pl.ds(base + _C, _C)], sidx1)
    pltpu.sync_copy(dst_hbm.at[pl.ds(base + _C, _C)], didx1)
    pltpu.async_copy(p_hbm.at[sidx0], rows0, g0)
    pltpu.async_copy(p_hbm.at[sidx1], rows1, g1)

    def pair(t, carry):
      a = 2 * t

      def bank(c, sidx, didx, rows, gsem, isem):
        pltpu.make_async_copy(p_hbm.at[sidx], rows, gsem).wait()
        pltpu.sync_copy(rows, acc_sh.at[didx], add=True)

        @pl.when(c + 2 < _NCH)
        def _next():
          off = base + (c + 2) * _C
          pltpu.async_copy(src_hbm.at[pl.ds(off, _C)], sidx, isem)
          pltpu.async_copy(dst_hbm.at[pl.ds(off, _C)], didx, isem)
          pltpu.make_async_copy(src_hbm.at[pl.ds(off, _C)], sidx, isem).wait()
          pltpu.make_async_copy(dst_hbm.at[pl.ds(off, _C)], didx, isem).wait()
          pltpu.async_copy(p_hbm.at[sidx], rows, gsem)

      bank(a, sidx0, didx0, rows0, g0, i0)
      bank(a + 1, sidx1, didx1, rows1, g1, i1)
      return carry

    lax.fori_loop(0, _NCH // 2, pair, 0)
    pltpu.make_async_copy(p_hbm.at[sidx0], rows0, g0).wait()
    pltpu.sync_copy(rows0, acc_sh.at[didx0], add=True)
    plsc.subcore_barrier()

    @pl.when(sid < _NS - 1)
    def _full():
      pltpu.sync_copy(
          acc_sh.at[pl.ds(sid * _RPS, _RPS)],
          out_hbm.at[cid, pl.ds(sid * _RPS, _RPS)],
      )

    @pl.when(sid == _NS - 1)
    def _tail():
      pltpu.sync_copy(
          acc_sh.at[pl.ds(sid * _RPS, _LAST)],
          out_hbm.at[cid, pl.ds(sid * _RPS, _LAST)],
      )

  return scat_kernel


def _dinv_block(dpt_ref):
  deg = dpt_ref[:, 0:1] + dpt_ref[:, 1:2] + 1.0
  return lax.rsqrt(deg)


def _tc_first(x, w, dpt):
  def body(x_ref, w_ref, dpt_ref, p_ref):
    dinv = _dinv_block(dpt_ref)
    h = jnp.dot(x_ref[...], w_ref[...], preferred_element_type=jnp.float32)
    p_ref[...] = h * dinv

  return pl.pallas_call(
      body,
      grid=(_TCG,),
      in_specs=[
          pl.BlockSpec((_TCB, _D), lambda i: (i, 0)),
          pl.BlockSpec((_D, _D), lambda i: (0, 0)),
          pl.BlockSpec((_TCB, _NC), lambda i: (i, 0)),
      ],
      out_specs=pl.BlockSpec((_TCB, _D), lambda i: (i, 0)),
      out_shape=jax.ShapeDtypeStruct((_N, _D), jnp.float32),
  )(x, w, dpt)


def _tc_mid(p, dpt, sp, w, b):
  def body(p_ref, dpt_ref, sp_ref, w_ref, b_ref, o_ref):
    dinv = _dinv_block(dpt_ref)
    s = sp_ref[0] + sp_ref[1] + p_ref[...]
    a = jnp.maximum(s * dinv + b_ref[...], 0.0)
    h = jnp.dot(a, w_ref[...], preferred_element_type=jnp.float32)
    o_ref[...] = h * dinv

  return pl.pallas_call(
      body,
      grid=(_TCG,),
      in_specs=[
          pl.BlockSpec((_TCB, _D), lambda i: (i, 0)),
          pl.BlockSpec((_TCB, _NC), lambda i: (i, 0)),
          pl.BlockSpec((_NC, _TCB, _D), lambda i: (0, i, 0)),
          pl.BlockSpec((_D, _D), lambda i: (0, 0)),
          pl.BlockSpec((1, _D), lambda i: (0, 0)),
      ],
      out_specs=pl.BlockSpec((_TCB, _D), lambda i: (i, 0)),
      out_shape=jax.ShapeDtypeStruct((_N, _D), jnp.float32),
  )(p, dpt, sp, w, b)


def _tc_final(p, dpt, sp, b):
  def body(p_ref, dpt_ref, sp_ref, b_ref, o_ref):
    dinv = _dinv_block(dpt_ref)
    s = sp_ref[0] + sp_ref[1] + p_ref[...]
    o_ref[...] = s * dinv + b_ref[...]

  return pl.pallas_call(
      body,
      grid=(_TCG,),
      in_specs=[
          pl.BlockSpec((_TCB, _D), lambda i: (i, 0)),
          pl.BlockSpec((_TCB, _NC), lambda i: (i, 0)),
          pl.BlockSpec((_NC, _TCB, _D), lambda i: (0, i, 0)),
          pl.BlockSpec((1, _D), lambda i: (0, 0)),
      ],
      out_specs=pl.BlockSpec((_TCB, _D), lambda i: (i, 0)),
      out_shape=jax.ShapeDtypeStruct((_N, _D), jnp.float32),
  )(p, dpt, sp, b)


_sc_degree = _make_sc_degree()
_sc_scatter = _make_sc_scatter()


def kernel(x, edge_index, W1, b1, W2, b2, W3, b3):
  src1 = edge_index[0]
  dst1 = edge_index[1]
  dst3 = dst1.reshape(_NW, _NCH, _C)
  b1r = b1.reshape(1, _D)
  b2r = b2.reshape(1, _D)
  b3r = b3.reshape(1, _D)

  dflat = _sc_degree(dst3).reshape(_NC, _NA)
  dpt = jnp.stack([dflat[0, :_N], dflat[1, :_N]], axis=1)
  p1 = _tc_first(x, W1, dpt)
  sp = _sc_scatter(p1, src1, dst1)
  p2 = _tc_mid(p1, dpt, sp, W2, b1r)
  sp = _sc_scatter(p2, src1, dst1)
  p3 = _tc_mid(p2, dpt, sp, W3, b2r)
  sp = _sc_scatter(p3, src1, dst1)
  return _tc_final(p3, dpt, sp, b3r)
